# baseline (device time: 67729 ns/iter reference)
import jax
import jax.numpy as jnp
from jax import lax
from jax.experimental import pallas as pl
from jax.experimental.pallas import tpu as pltpu

N_DEV = 8
B, Sq, Hq, Hkv, Dh = 2, 256, 8, 2, 64
G = Hq // Hkv
SCALE = 0.125


def kernel(x, Wq, Wo, K_ext, V_ext):
    skv = K_ext.shape[1]
    k2 = K_ext.reshape(B, skv, Hkv * Dh)
    v2 = V_ext.reshape(B, skv, Hkv * Dh)

    def body(x_ref, wq_ref, wo_ref, k_ref, v_ref, out_ref,
             commk, commv, ksend, krecv, vsend, vrecv):
        my = lax.axis_index("i")
        right = lax.rem(my + 1, N_DEV)
        left = lax.rem(my + N_DEV - 1, N_DEV)

        barrier = pltpu.get_barrier_semaphore()
        pl.semaphore_signal(barrier, inc=1, device_id=(left,),
                            device_id_type=pl.DeviceIdType.MESH)
        pl.semaphore_signal(barrier, inc=1, device_id=(right,),
                            device_id_type=pl.DeviceIdType.MESH)
        pl.semaphore_wait(barrier, 2)

        commk[0] = k_ref[...].astype(jnp.bfloat16)
        commv[0] = v_ref[...].astype(jnp.bfloat16)

        wq = wq_ref[...].astype(jnp.bfloat16)
        qf = []
        for b in range(B):
            qb = jnp.dot(x_ref[b].astype(jnp.bfloat16), wq,
                         preferred_element_type=jnp.float32)
            qb = qb.astype(jnp.bfloat16)
            qf.append([
                jnp.concatenate(
                    [qb[:, (g * G + hh) * Dh:(g * G + hh + 1) * Dh]
                     for hh in range(G)], axis=0)
                for g in range(Hkv)
            ])

        m = [[jnp.full((G * Sq, 1), -jnp.inf, jnp.float32)
              for _ in range(Hkv)] for _ in range(B)]
        l = [[jnp.zeros((G * Sq, 1), jnp.float32)
              for _ in range(Hkv)] for _ in range(B)]
        acc = [[jnp.zeros((G * Sq, Dh), jnp.float32)
                for _ in range(Hkv)] for _ in range(B)]

        def process(slot):
            kc = commk[slot]
            vc = commv[slot]
            for b in range(B):
                for g in range(Hkv):
                    kbg = kc[b, :, g * Dh:(g + 1) * Dh]
                    vbg = vc[b, :, g * Dh:(g + 1) * Dh]
                    s = lax.dot_general(
                        qf[b][g], kbg, (((1,), (1,)), ((), ())),
                        preferred_element_type=jnp.float32) * SCALE
                    mj = jnp.max(s, axis=1, keepdims=True)
                    m_new = jnp.maximum(m[b][g], mj)
                    alpha = jnp.exp(m[b][g] - m_new)
                    p = jnp.exp(s - m_new)
                    l[b][g] = l[b][g] * alpha + jnp.sum(p, axis=1, keepdims=True)
                    acc[b][g] = acc[b][g] * alpha + jnp.dot(
                        p.astype(jnp.bfloat16), vbg,
                        preferred_element_type=jnp.float32)
                    m[b][g] = m_new

        for h in range(N_DEV):
            if h < N_DEV - 1:
                rk = pltpu.make_async_remote_copy(
                    src_ref=commk.at[h], dst_ref=commk.at[h + 1],
                    send_sem=ksend.at[h], recv_sem=krecv.at[h],
                    device_id=(right,), device_id_type=pl.DeviceIdType.MESH)
                rv = pltpu.make_async_remote_copy(
                    src_ref=commv.at[h], dst_ref=commv.at[h + 1],
                    send_sem=vsend.at[h], recv_sem=vrecv.at[h],
                    device_id=(right,), device_id_type=pl.DeviceIdType.MESH)
                rk.start()
                rv.start()
            process(h)
            if h < N_DEV - 1:
                rk.wait()
                rv.wait()

        wo = wo_ref[...].astype(jnp.bfloat16)
        for b in range(B):
            cols = []
            for g in range(Hkv):
                o = acc[b][g] / l[b][g]
                for hh in range(G):
                    cols.append(o[hh * Sq:(hh + 1) * Sq, :])
            ob = jnp.concatenate(cols, axis=1).astype(jnp.bfloat16)
            out_ref[b] = jnp.dot(ob, wo, preferred_element_type=jnp.float32)

    return pl.pallas_call(
        body,
        out_shape=jax.ShapeDtypeStruct((B, Sq, 768), jnp.float32),
        in_specs=[pl.BlockSpec(memory_space=pltpu.VMEM)] * 5,
        out_specs=pl.BlockSpec(memory_space=pltpu.VMEM),
        scratch_shapes=[
            pltpu.VMEM((N_DEV, B, skv, Hkv * Dh), jnp.bfloat16),
            pltpu.VMEM((N_DEV, B, skv, Hkv * Dh), jnp.bfloat16),
            pltpu.SemaphoreType.DMA((N_DEV - 1,)),
            pltpu.SemaphoreType.DMA((N_DEV - 1,)),
            pltpu.SemaphoreType.DMA((N_DEV - 1,)),
            pltpu.SemaphoreType.DMA((N_DEV - 1,)),
        ],
        compiler_params=pltpu.CompilerParams(collective_id=0),
    )(x, Wq, Wo, k2, v2)


# device time: 46798 ns/iter; 1.4473x vs baseline; 1.4473x over previous
import jax
import jax.numpy as jnp
from jax import lax
from jax.experimental import pallas as pl
from jax.experimental.pallas import tpu as pltpu

N_DEV = 8
B, Sq, Hq, Hkv, Dh = 2, 256, 8, 2, 64
G = Hq // Hkv
SCALE = 0.125
KV = Hkv * Dh
HOPS_A = N_DEV // 2
HOPS_B = N_DEV - 1 - HOPS_A


def kernel(x, Wq, Wo, K_ext, V_ext):
    skv = K_ext.shape[1]
    k2 = K_ext.reshape(B, skv, KV)
    v2 = V_ext.reshape(B, skv, KV)

    def body(x_ref, wq_ref, wo_ref, k_ref, v_ref, out_ref,
             commA, commB, sa_send, sa_recv, sb_send, sb_recv):
        my = lax.axis_index("i")
        right = lax.rem(my + 1, N_DEV)
        left = lax.rem(my + N_DEV - 1, N_DEV)

        commA[0] = jnp.concatenate(
            [k_ref[...], v_ref[...]], axis=-1).astype(jnp.bfloat16)

        barrier = pltpu.get_barrier_semaphore()
        pl.semaphore_signal(barrier, inc=1, device_id=(left,),
                            device_id_type=pl.DeviceIdType.MESH)
        pl.semaphore_signal(barrier, inc=1, device_id=(right,),
                            device_id_type=pl.DeviceIdType.MESH)
        pl.semaphore_wait(barrier, 2)

        def send_right(h):
            r = pltpu.make_async_remote_copy(
                src_ref=commA.at[h], dst_ref=commA.at[h + 1],
                send_sem=sa_send.at[h], recv_sem=sa_recv.at[h],
                device_id=(right,), device_id_type=pl.DeviceIdType.MESH)
            r.start()
            return r

        def send_left(h):
            src = commA.at[0] if h == 0 else commB.at[h]
            r = pltpu.make_async_remote_copy(
                src_ref=src, dst_ref=commB.at[h + 1],
                send_sem=sb_send.at[h], recv_sem=sb_recv.at[h],
                device_id=(left,), device_id_type=pl.DeviceIdType.MESH)
            r.start()
            return r

        ra = send_right(0)
        rb = send_left(0)

        wq = wq_ref[...].astype(jnp.bfloat16)
        qf = []
        for b in range(B):
            qb = jnp.dot(x_ref[b].astype(jnp.bfloat16), wq,
                         preferred_element_type=jnp.float32)
            qb = qb.astype(jnp.bfloat16)
            qf.append([
                jnp.concatenate(
                    [qb[:, (g * G + hh) * Dh:(g * G + hh + 1) * Dh]
                     for hh in range(G)], axis=0)
                for g in range(Hkv)
            ])

        m = [[jnp.full((G * Sq, 1), -jnp.inf, jnp.float32)
              for _ in range(Hkv)] for _ in range(B)]
        l = [[jnp.zeros((G * Sq, 1), jnp.float32)
              for _ in range(Hkv)] for _ in range(B)]
        acc = [[jnp.zeros((G * Sq, Dh), jnp.float32)
                for _ in range(Hkv)] for _ in range(B)]

        def process(buf, slot):
            c = buf[slot]
            for b in range(B):
                for g in range(Hkv):
                    kbg = c[b, :, g * Dh:(g + 1) * Dh]
                    vbg = c[b, :, KV + g * Dh:KV + (g + 1) * Dh]
                    s = lax.dot_general(
                        qf[b][g], kbg, (((1,), (1,)), ((), ())),
                        preferred_element_type=jnp.float32) * SCALE
                    mj = jnp.max(s, axis=1, keepdims=True)
                    m_new = jnp.maximum(m[b][g], mj)
                    alpha = jnp.exp(m[b][g] - m_new)
                    p = jnp.exp(s - m_new)
                    l[b][g] = l[b][g] * alpha + jnp.sum(p, axis=1, keepdims=True)
                    acc[b][g] = acc[b][g] * alpha + jnp.dot(
                        p.astype(jnp.bfloat16), vbg,
                        preferred_element_type=jnp.float32)
                    m[b][g] = m_new

        process(commA, 0)
        ra.wait()
        rb.wait()

        for step in range(1, HOPS_A):
            ra = send_right(step)
            rb = send_left(step) if step < HOPS_B else None
            process(commA, step)
            if step <= HOPS_B:
                process(commB, step)
            ra.wait()
            if rb is not None:
                rb.wait()
        process(commA, HOPS_A)

        wo = wo_ref[...].astype(jnp.bfloat16)
        for b in range(B):
            cols = []
            for g in range(Hkv):
                o = acc[b][g] / l[b][g]
                for hh in range(G):
                    cols.append(o[hh * Sq:(hh + 1) * Sq, :])
            ob = jnp.concatenate(cols, axis=1).astype(jnp.bfloat16)
            out_ref[b] = jnp.dot(ob, wo, preferred_element_type=jnp.float32)

    return pl.pallas_call(
        body,
        out_shape=jax.ShapeDtypeStruct((B, Sq, 768), jnp.float32),
        in_specs=[pl.BlockSpec(memory_space=pltpu.VMEM)] * 5,
        out_specs=pl.BlockSpec(memory_space=pltpu.VMEM),
        scratch_shapes=[
            pltpu.VMEM((HOPS_A + 1, B, skv, 2 * KV), jnp.bfloat16),
            pltpu.VMEM((HOPS_B + 1, B, skv, 2 * KV), jnp.bfloat16),
            pltpu.SemaphoreType.DMA((HOPS_A,)),
            pltpu.SemaphoreType.DMA((HOPS_A,)),
            pltpu.SemaphoreType.DMA((HOPS_B,)),
            pltpu.SemaphoreType.DMA((HOPS_B,)),
        ],
        compiler_params=pltpu.CompilerParams(collective_id=0),
    )(x, Wq, Wo, k2, v2)


# device time: 45284 ns/iter; 1.4956x vs baseline; 1.0334x over previous
import jax
import jax.numpy as jnp
from jax import lax
from jax.experimental import pallas as pl
from jax.experimental.pallas import tpu as pltpu

N_DEV = 8
B, Sq, Hq, Hkv, Dh = 2, 256, 8, 2, 64
G = Hq // Hkv
SCALE = 0.125
KV = Hkv * Dh
HOPS_A = N_DEV // 2
HOPS_B = N_DEV - 1 - HOPS_A


def kernel(x, Wq, Wo, K_ext, V_ext):
    skv = K_ext.shape[1]
    k2 = K_ext.reshape(B, skv, KV)
    v2 = V_ext.reshape(B, skv, KV)

    def body(x_ref, wq_ref, wo_ref, k_ref, v_ref, out_ref,
             commA, commB, sa_send, sa_recv, sb_send, sb_recv):
        my = lax.axis_index("i")
        right = lax.rem(my + 1, N_DEV)
        left = lax.rem(my + N_DEV - 1, N_DEV)

        commA[0] = jnp.concatenate(
            [k_ref[...], v_ref[...]], axis=-1).astype(jnp.bfloat16)

        barrier = pltpu.get_barrier_semaphore()
        pl.semaphore_signal(barrier, inc=1, device_id=(left,),
                            device_id_type=pl.DeviceIdType.MESH)
        pl.semaphore_signal(barrier, inc=1, device_id=(right,),
                            device_id_type=pl.DeviceIdType.MESH)
        pl.semaphore_wait(barrier, 2)

        def send_right(h):
            r = pltpu.make_async_remote_copy(
                src_ref=commA.at[h], dst_ref=commA.at[h + 1],
                send_sem=sa_send.at[h], recv_sem=sa_recv.at[h],
                device_id=(right,), device_id_type=pl.DeviceIdType.MESH)
            r.start()
            return r

        def send_left(h):
            src = commA.at[0] if h == 0 else commB.at[h]
            r = pltpu.make_async_remote_copy(
                src_ref=src, dst_ref=commB.at[h + 1],
                send_sem=sb_send.at[h], recv_sem=sb_recv.at[h],
                device_id=(left,), device_id_type=pl.DeviceIdType.MESH)
            r.start()
            return r

        ra = send_right(0)
        rb = send_left(0)

        wq = wq_ref[...].astype(jnp.bfloat16)
        qf = []
        for b in range(B):
            qb = jnp.dot(x_ref[b].astype(jnp.bfloat16), wq,
                         preferred_element_type=jnp.float32)
            qb = (qb * SCALE).astype(jnp.bfloat16)
            qf.append([
                jnp.concatenate(
                    [qb[:, (g * G + hh) * Dh:(g * G + hh + 1) * Dh]
                     for hh in range(G)], axis=0)
                for g in range(Hkv)
            ])

        accl = [[jnp.zeros((G * Sq, Dh + 1), jnp.float32)
                 for _ in range(Hkv)] for _ in range(B)]
        ones_col = jnp.ones((skv, 1), jnp.bfloat16)

        def process(buf, slot):
            c = buf[slot]
            for b in range(B):
                for g in range(Hkv):
                    kbg = c[b, :, g * Dh:(g + 1) * Dh]
                    vbg = c[b, :, KV + g * Dh:KV + (g + 1) * Dh]
                    s = lax.dot_general(
                        qf[b][g], kbg, (((1,), (1,)), ((), ())),
                        preferred_element_type=jnp.float32)
                    p = jnp.exp(s).astype(jnp.bfloat16)
                    v_ext = jnp.concatenate([vbg, ones_col], axis=1)
                    accl[b][g] = accl[b][g] + jnp.dot(
                        p, v_ext, preferred_element_type=jnp.float32)

        process(commA, 0)
        ra.wait()
        rb.wait()

        for step in range(1, HOPS_A):
            ra = send_right(step)
            rb = send_left(step) if step < HOPS_B else None
            process(commA, step)
            if step <= HOPS_B:
                process(commB, step)
            ra.wait()
            if rb is not None:
                rb.wait()
        process(commA, HOPS_A)

        wo = wo_ref[...].astype(jnp.bfloat16)
        for b in range(B):
            cols = []
            for g in range(Hkv):
                o = accl[b][g][:, :Dh] / accl[b][g][:, Dh:]
                for hh in range(G):
                    cols.append(o[hh * Sq:(hh + 1) * Sq, :])
            ob = jnp.concatenate(cols, axis=1).astype(jnp.bfloat16)
            out_ref[b] = jnp.dot(ob, wo, preferred_element_type=jnp.float32)

    return pl.pallas_call(
        body,
        out_shape=jax.ShapeDtypeStruct((B, Sq, 768), jnp.float32),
        in_specs=[pl.BlockSpec(memory_space=pltpu.VMEM)] * 5,
        out_specs=pl.BlockSpec(memory_space=pltpu.VMEM),
        scratch_shapes=[
            pltpu.VMEM((HOPS_A + 1, B, skv, 2 * KV), jnp.bfloat16),
            pltpu.VMEM((HOPS_B + 1, B, skv, 2 * KV), jnp.bfloat16),
            pltpu.SemaphoreType.DMA((HOPS_A,)),
            pltpu.SemaphoreType.DMA((HOPS_A,)),
            pltpu.SemaphoreType.DMA((HOPS_B,)),
            pltpu.SemaphoreType.DMA((HOPS_B,)),
        ],
        compiler_params=pltpu.CompilerParams(collective_id=0),
    )(x, Wq, Wo, k2, v2)


# device time: 37310 ns/iter; 1.8153x vs baseline; 1.2137x over previous
import jax
import jax.numpy as jnp
from jax import lax
from jax.experimental import pallas as pl
from jax.experimental.pallas import tpu as pltpu

N_DEV = 8
B, Sq, Hq, Hkv, Dh = 2, 256, 8, 2, 64
G = Hq // Hkv
SCALE = 0.125
KV = Hkv * Dh

_ROUND1 = [(0, 1, 0), (0, 2, 1), (0, 3, 2)]
_ROUND2 = [(2, 4, 0), (3, 5, 1), (1, 6, 2)]
_ROUND3 = [(5, 7, 0)]


def kernel(x, Wq, Wo, K_ext, V_ext):
    skv = K_ext.shape[1]
    k2 = K_ext.reshape(B, skv, KV)
    v2 = V_ext.reshape(B, skv, KV)

    def body(x_ref, wq_ref, wo_ref, k_ref, v_ref, out_ref,
             buf, send_sems, recv_sems):
        my = lax.axis_index("i")
        nx = jnp.where(my < 4, 3 - my, 11 - my)
        ny = jnp.where(lax.rem(my, 2) == 0, my + 1, my - 1)
        nz = lax.rem(my + 4, N_DEV)
        nbr = [nx, ny, nz]

        buf[0] = jnp.concatenate(
            [k_ref[...], v_ref[...]], axis=-1).astype(jnp.bfloat16)

        barrier = pltpu.get_barrier_semaphore()
        for n in nbr:
            pl.semaphore_signal(barrier, inc=1, device_id=(n,),
                                device_id_type=pl.DeviceIdType.MESH)
        pl.semaphore_wait(barrier, 3)

        sem_idx = [0]

        def start(transfers):
            rs = []
            for src, dst, axis in transfers:
                i = sem_idx[0]
                sem_idx[0] += 1
                r = pltpu.make_async_remote_copy(
                    src_ref=buf.at[src], dst_ref=buf.at[dst],
                    send_sem=send_sems.at[i], recv_sem=recv_sems.at[i],
                    device_id=(nbr[axis],),
                    device_id_type=pl.DeviceIdType.MESH)
                r.start()
                rs.append(r)
            return rs

        rs = start(_ROUND1)

        wq = wq_ref[...].astype(jnp.bfloat16)
        qf = []
        for b in range(B):
            qb = jnp.dot(x_ref[b].astype(jnp.bfloat16), wq,
                         preferred_element_type=jnp.float32)
            qb = (qb * SCALE).astype(jnp.bfloat16)
            qf.append([
                jnp.concatenate(
                    [qb[:, (g * G + hh) * Dh:(g * G + hh + 1) * Dh]
                     for hh in range(G)], axis=0)
                for g in range(Hkv)
            ])

        accl = [[jnp.zeros((G * Sq, Dh + 1), jnp.float32)
                 for _ in range(Hkv)] for _ in range(B)]
        ones_col = jnp.ones((skv, 1), jnp.bfloat16)

        def process(slot):
            c = buf[slot]
            for b in range(B):
                for g in range(Hkv):
                    kbg = c[b, :, g * Dh:(g + 1) * Dh]
                    vbg = c[b, :, KV + g * Dh:KV + (g + 1) * Dh]
                    s = lax.dot_general(
                        qf[b][g], kbg, (((1,), (1,)), ((), ())),
                        preferred_element_type=jnp.float32)
                    p = jnp.exp(s).astype(jnp.bfloat16)
                    v_ext = jnp.concatenate([vbg, ones_col], axis=1)
                    accl[b][g] = accl[b][g] + jnp.dot(
                        p, v_ext, preferred_element_type=jnp.float32)

        process(0)
        for r in rs:
            r.wait()

        rs = start(_ROUND2)
        for slot in (1, 2, 3):
            process(slot)
        for r in rs:
            r.wait()

        rs = start(_ROUND3)
        for slot in (4, 5, 6):
            process(slot)
        for r in rs:
            r.wait()
        process(7)

        wo = wo_ref[...].astype(jnp.bfloat16)
        for b in range(B):
            cols = []
            for g in range(Hkv):
                o = accl[b][g][:, :Dh] / accl[b][g][:, Dh:]
                for hh in range(G):
                    cols.append(o[hh * Sq:(hh + 1) * Sq, :])
            ob = jnp.concatenate(cols, axis=1).astype(jnp.bfloat16)
            out_ref[b] = jnp.dot(ob, wo, preferred_element_type=jnp.float32)

    return pl.pallas_call(
        body,
        out_shape=jax.ShapeDtypeStruct((B, Sq, 768), jnp.float32),
        in_specs=[pl.BlockSpec(memory_space=pltpu.VMEM)] * 5,
        out_specs=pl.BlockSpec(memory_space=pltpu.VMEM),
        scratch_shapes=[
            pltpu.VMEM((N_DEV, B, skv, 2 * KV), jnp.bfloat16),
            pltpu.SemaphoreType.DMA((7,)),
            pltpu.SemaphoreType.DMA((7,)),
        ],
        compiler_params=pltpu.CompilerParams(collective_id=0),
    )(x, Wq, Wo, k2, v2)
